# baseline (speedup 1.0000x reference)
import functools

import jax
import jax.numpy as jnp
from jax import lax
from jax.experimental import pallas as pl
from jax.experimental.pallas import tpu as pltpu
from jax.experimental.pallas import tpu_sc as plsc

N = 10000
E = 320000
D = 128
NC = 2
NS = 16
NW = NC * NS
CHUNK = 128
K0 = 48
K1 = 112
KMAX = max(K0, K1)
TOT_CHUNKS = NS * (K0 + K1)
E_PAD = TOT_CHUNKS * CHUNK
AGG_ROWS = 10016
ZROWS = 632
ZROWS_LAST = AGG_ROWS - (NS - 1) * ZROWS
OROWS = 632
OROWS_LAST = N - (NS - 1) * OROWS


def _sc_aggregate(h, zeros, src, dst):
    mesh = plsc.VectorSubcoreMesh(core_axis_name="c", subcore_axis_name="s",
                                  num_cores=NC, num_subcores=NS)

    @functools.partial(
        pl.kernel,
        out_type=jax.ShapeDtypeStruct((NC, N, D), jnp.float32),
        mesh=mesh,
        scratch_types=[
            pltpu.VMEM((KMAX, CHUNK), jnp.int32),
            pltpu.VMEM((CHUNK,), jnp.int32),
            pltpu.VMEM((CHUNK,), jnp.int32),
            pltpu.VMEM((CHUNK, D), jnp.float32),
            pltpu.VMEM((CHUNK, D), jnp.float32),
            pltpu.SemaphoreType.DMA,
            pltpu.SemaphoreType.DMA,
            pltpu.SemaphoreType.DMA,
            pltpu.SemaphoreType.DMA,
            pltpu.VMEM_SHARED((AGG_ROWS, D), jnp.float32),
        ],
    )
    def agg_kernel(h_hbm, z_hbm, src_hbm, dst_hbm, out_hbm,
                   src_v, dst0, dst1, rows0, rows1,
                   gsem0, gsem1, dsem0, dsem1, acc_sh):
        c = lax.axis_index("c")
        s = lax.axis_index("s")
        base_chunk = jnp.where(c == 0, s * K0, NS * K0 + s * K1)
        kc = jnp.where(c == 0, K0, K1)

        @pl.when(s < NS - 1)
        def _():
            pltpu.sync_copy(z_hbm.at[pl.ds(0, ZROWS)],
                            acc_sh.at[pl.ds(s * ZROWS, ZROWS)])

        @pl.when(s == NS - 1)
        def _():
            pltpu.sync_copy(z_hbm.at[pl.ds(0, ZROWS_LAST)],
                            acc_sh.at[pl.ds((NS - 1) * ZROWS, ZROWS_LAST)])

        @pl.when(c == 0)
        def _():
            pltpu.sync_copy(src_hbm.at[pl.ds(s * K0, K0)],
                            src_v.at[pl.ds(0, K0)])

        @pl.when(c == 1)
        def _():
            pltpu.sync_copy(src_hbm.at[pl.ds(NS * K0 + s * K1, K1)],
                            src_v.at[pl.ds(0, K1)])

        plsc.subcore_barrier()

        pltpu.async_copy(dst_hbm.at[base_chunk], dst0, dsem0)
        pltpu.async_copy(h_hbm.at[src_v.at[0]], rows0, gsem0)

        def step(k, dst_a, dsem_a, rows_a, gsem_a, dst_b, dsem_b, rows_b, gsem_b):
            @pl.when(k + 1 < kc)
            def _():
                pltpu.async_copy(dst_hbm.at[base_chunk + k + 1], dst_b, dsem_b)
                pltpu.async_copy(h_hbm.at[src_v.at[k + 1]], rows_b, gsem_b)

            pltpu.make_async_copy(dst_hbm.at[base_chunk + k], dst_a, dsem_a).wait()
            pltpu.make_async_copy(h_hbm.at[src_v.at[k]], rows_a, gsem_a).wait()
            pltpu.sync_copy(rows_a, acc_sh.at[dst_a], add=True)

        def body(k, carry):
            @pl.when(lax.rem(k, 2) == 0)
            def _():
                step(k, dst0, dsem0, rows0, gsem0, dst1, dsem1, rows1, gsem1)

            @pl.when(lax.rem(k, 2) == 1)
            def _():
                step(k, dst1, dsem1, rows1, gsem1, dst0, dsem0, rows0, gsem0)

            return carry

        lax.fori_loop(0, kc, body, 0)
        plsc.subcore_barrier()

        @pl.when(s < NS - 1)
        def _():
            pltpu.sync_copy(acc_sh.at[pl.ds(s * OROWS, OROWS)],
                            out_hbm.at[c, pl.ds(s * OROWS, OROWS)])

        @pl.when(s == NS - 1)
        def _():
            pltpu.sync_copy(acc_sh.at[pl.ds((NS - 1) * OROWS, OROWS_LAST)],
                            out_hbm.at[c, pl.ds((NS - 1) * OROWS, OROWS_LAST)])

    return agg_kernel(h, zeros, src, dst)


def _tc_linear(parts, w_t, b, relu):
    blk = 1000
    grid = N // blk

    def body(*refs):
        *a_refs, w_ref, b_ref, o_ref = refs
        h = a_refs[0][...]
        for r in a_refs[1:]:
            h = h + r[...]
        y = jnp.dot(h, w_ref[...], preferred_element_type=jnp.float32) + b_ref[...]
        if relu:
            y = jnp.maximum(y, 0.0)
        o_ref[...] = y

    in_specs = [pl.BlockSpec((blk, D), lambda i: (i, 0)) for _ in parts]
    in_specs += [
        pl.BlockSpec((D, D), lambda i: (0, 0)),
        pl.BlockSpec((1, D), lambda i: (0, 0)),
    ]
    return pl.pallas_call(
        body,
        grid=(grid,),
        in_specs=in_specs,
        out_specs=pl.BlockSpec((blk, D), lambda i: (i, 0)),
        out_shape=jax.ShapeDtypeStruct((N, D), jnp.float32),
    )(*parts, w_t, b)


def kernel(x, edge_index, W_conv, b_conv, W_out, b_out):
    ei = edge_index.astype(jnp.int32)
    pad = E_PAD - E
    src = jnp.concatenate([ei[0], jnp.zeros((pad,), jnp.int32)])
    dst = jnp.concatenate([ei[1], jnp.full((pad,), N, jnp.int32)])
    src = src.reshape(TOT_CHUNKS, CHUNK)
    dst = dst.reshape(TOT_CHUNKS, CHUNK)
    zeros = jnp.zeros((ZROWS, D), jnp.float32)
    wc_t = W_conv.T
    wo_t = W_out.T
    bc = b_conv.reshape(1, D)
    bo = b_out.reshape(1, D)

    h = x
    for _ in range(2):
        parts = _sc_aggregate(h, zeros, src, dst)
        h = _tc_linear((parts[0], parts[1]), wc_t, bc, relu=True)
    return _tc_linear((h,), wo_t, bo, relu=False)


# speedup vs baseline: 1.0684x; 1.0684x over previous
import functools

import jax
import jax.numpy as jnp
from jax import lax
from jax.experimental import pallas as pl
from jax.experimental.pallas import tpu as pltpu
from jax.experimental.pallas import tpu_sc as plsc

N = 10000
E = 320000
D = 128
NC = 2
NS = 16
NW = NC * NS
CHUNK = 128
K0 = 112
K1 = 48
KMAX = max(K0, K1)
TOT_CHUNKS = NS * (K0 + K1)
E_PAD = TOT_CHUNKS * CHUNK
AGG_ROWS = 10016
ZROWS = 632
ZROWS_LAST = AGG_ROWS - (NS - 1) * ZROWS
OROWS = 632
OROWS_LAST = N - (NS - 1) * OROWS


def _sc_aggregate(h, zeros, src, dst):
    mesh = plsc.VectorSubcoreMesh(core_axis_name="c", subcore_axis_name="s",
                                  num_cores=NC, num_subcores=NS)

    @functools.partial(
        pl.kernel,
        out_type=jax.ShapeDtypeStruct((NC, N, D), jnp.float32),
        mesh=mesh,
        scratch_types=[
            pltpu.VMEM((KMAX, CHUNK), jnp.int32),
            pltpu.VMEM((CHUNK,), jnp.int32),
            pltpu.VMEM((CHUNK,), jnp.int32),
            pltpu.VMEM((CHUNK, D), jnp.float32),
            pltpu.VMEM((CHUNK, D), jnp.float32),
            pltpu.SemaphoreType.DMA,
            pltpu.SemaphoreType.DMA,
            pltpu.SemaphoreType.DMA,
            pltpu.SemaphoreType.DMA,
            pltpu.VMEM_SHARED((AGG_ROWS, D), jnp.float32),
        ],
    )
    def agg_kernel(h_hbm, z_hbm, src_hbm, dst_hbm, out_hbm,
                   src_v, dst0, dst1, rows0, rows1,
                   gsem0, gsem1, dsem0, dsem1, acc_sh):
        c = lax.axis_index("c")
        s = lax.axis_index("s")
        base_chunk = jnp.where(c == 0, s * K0, NS * K0 + s * K1)
        kc = jnp.where(c == 0, K0, K1)

        @pl.when(s < NS - 1)
        def _():
            pltpu.sync_copy(z_hbm.at[pl.ds(0, ZROWS)],
                            acc_sh.at[pl.ds(s * ZROWS, ZROWS)])

        @pl.when(s == NS - 1)
        def _():
            pltpu.sync_copy(z_hbm.at[pl.ds(0, ZROWS_LAST)],
                            acc_sh.at[pl.ds((NS - 1) * ZROWS, ZROWS_LAST)])

        @pl.when(c == 0)
        def _():
            pltpu.sync_copy(src_hbm.at[pl.ds(s * K0, K0)],
                            src_v.at[pl.ds(0, K0)])

        @pl.when(c == 1)
        def _():
            pltpu.sync_copy(src_hbm.at[pl.ds(NS * K0 + s * K1, K1)],
                            src_v.at[pl.ds(0, K1)])

        plsc.subcore_barrier()

        pltpu.async_copy(dst_hbm.at[base_chunk], dst0, dsem0)
        pltpu.async_copy(h_hbm.at[src_v.at[0]], rows0, gsem0)

        def step(k, dst_a, dsem_a, rows_a, gsem_a, dst_b, dsem_b, rows_b, gsem_b):
            @pl.when(k + 1 < kc)
            def _():
                pltpu.async_copy(dst_hbm.at[base_chunk + k + 1], dst_b, dsem_b)
                pltpu.async_copy(h_hbm.at[src_v.at[k + 1]], rows_b, gsem_b)

            pltpu.make_async_copy(dst_hbm.at[base_chunk + k], dst_a, dsem_a).wait()
            pltpu.make_async_copy(h_hbm.at[src_v.at[k]], rows_a, gsem_a).wait()
            pltpu.sync_copy(rows_a, acc_sh.at[dst_a], add=True)

        def body(k, carry):
            @pl.when(lax.rem(k, 2) == 0)
            def _():
                step(k, dst0, dsem0, rows0, gsem0, dst1, dsem1, rows1, gsem1)

            @pl.when(lax.rem(k, 2) == 1)
            def _():
                step(k, dst1, dsem1, rows1, gsem1, dst0, dsem0, rows0, gsem0)

            return carry

        lax.fori_loop(0, kc, body, 0)
        plsc.subcore_barrier()

        @pl.when(s < NS - 1)
        def _():
            pltpu.sync_copy(acc_sh.at[pl.ds(s * OROWS, OROWS)],
                            out_hbm.at[c, pl.ds(s * OROWS, OROWS)])

        @pl.when(s == NS - 1)
        def _():
            pltpu.sync_copy(acc_sh.at[pl.ds((NS - 1) * OROWS, OROWS_LAST)],
                            out_hbm.at[c, pl.ds((NS - 1) * OROWS, OROWS_LAST)])

    return agg_kernel(h, zeros, src, dst)


def _tc_linear(parts, w_t, b, relu):
    blk = 1000
    grid = N // blk

    def body(*refs):
        *a_refs, w_ref, b_ref, o_ref = refs
        h = a_refs[0][...]
        for r in a_refs[1:]:
            h = h + r[...]
        y = jnp.dot(h, w_ref[...], preferred_element_type=jnp.float32) + b_ref[...]
        if relu:
            y = jnp.maximum(y, 0.0)
        o_ref[...] = y

    in_specs = [pl.BlockSpec((blk, D), lambda i: (i, 0)) for _ in parts]
    in_specs += [
        pl.BlockSpec((D, D), lambda i: (0, 0)),
        pl.BlockSpec((1, D), lambda i: (0, 0)),
    ]
    return pl.pallas_call(
        body,
        grid=(grid,),
        in_specs=in_specs,
        out_specs=pl.BlockSpec((blk, D), lambda i: (i, 0)),
        out_shape=jax.ShapeDtypeStruct((N, D), jnp.float32),
    )(*parts, w_t, b)


def kernel(x, edge_index, W_conv, b_conv, W_out, b_out):
    ei = edge_index.astype(jnp.int32)
    pad = E_PAD - E
    src = jnp.concatenate([ei[0], jnp.zeros((pad,), jnp.int32)])
    dst = jnp.concatenate([ei[1], jnp.full((pad,), N, jnp.int32)])
    src = src.reshape(TOT_CHUNKS, CHUNK)
    dst = dst.reshape(TOT_CHUNKS, CHUNK)
    zeros = jnp.zeros((ZROWS, D), jnp.float32)
    wc_t = W_conv.T
    wo_t = W_out.T
    bc = b_conv.reshape(1, D)
    bo = b_out.reshape(1, D)

    h = x
    for _ in range(2):
        parts = _sc_aggregate(h, zeros, src, dst)
        h = _tc_linear((parts[0], parts[1]), wc_t, bc, relu=True)
    return _tc_linear((h,), wo_t, bo, relu=False)


# speedup vs baseline: 1.7393x; 1.6279x over previous
import functools

import jax
import jax.numpy as jnp
from jax import lax
from jax.experimental import pallas as pl
from jax.experimental.pallas import tpu as pltpu
from jax.experimental.pallas import tpu_sc as plsc

N = 10000
E = 320000
D = 128
NC = 2
NS = 16
NW = NC * NS
CHUNK = 128
KTILE = 79
E_PAD = KTILE * NW * CHUNK
TOT_CHUNKS = KTILE * NW
AGG_ROWS = 10016
ZROWS = 632
ZROWS_LAST = AGG_ROWS - (NS - 1) * ZROWS
OROWS = 632
OROWS_LAST = N - (NS - 1) * OROWS


def _sc_aggregate(h, zeros, src, dst):
    mesh = plsc.VectorSubcoreMesh(core_axis_name="c", subcore_axis_name="s",
                                  num_cores=NC, num_subcores=NS)

    @functools.partial(
        pl.kernel,
        out_type=jax.ShapeDtypeStruct((NC, N, D), jnp.float32),
        mesh=mesh,
        scratch_types=[
            [pltpu.VMEM((CHUNK,), jnp.int32) for _ in range(3)],
            [pltpu.VMEM((CHUNK,), jnp.int32) for _ in range(3)],
            [pltpu.VMEM((CHUNK, D), jnp.float32) for _ in range(3)],
            [pltpu.SemaphoreType.DMA for _ in range(9)],
            pltpu.VMEM_SHARED((AGG_ROWS, D), jnp.float32),
        ],
    )
    def agg_kernel(h_hbm, z_hbm, src_hbm, dst_hbm, out_hbm,
                   srcb, dstb, rows, sems, acc_sh):
        ssem, dsem, gsem = sems[0:3], sems[3:6], sems[6:9]
        c = lax.axis_index("c")
        s = lax.axis_index("s")

        @pl.when(s < NS - 1)
        def _():
            pltpu.sync_copy(z_hbm.at[pl.ds(0, ZROWS)],
                            acc_sh.at[pl.ds(s * ZROWS, ZROWS)])

        @pl.when(s == NS - 1)
        def _():
            pltpu.sync_copy(z_hbm.at[pl.ds(0, ZROWS_LAST)],
                            acc_sh.at[pl.ds((NS - 1) * ZROWS, ZROWS_LAST)])

        plsc.subcore_barrier()

        base = (c * NS + s) * KTILE

        for j in range(3):
            pltpu.async_copy(src_hbm.at[base + j], srcb[j], ssem[j])
            pltpu.async_copy(dst_hbm.at[base + j], dstb[j], dsem[j])
        for j in range(2):
            pltpu.make_async_copy(src_hbm.at[base + j], srcb[j], ssem[j]).wait()
            pltpu.async_copy(h_hbm.at[srcb[j]], rows[j], gsem[j])

        def step(k, a, b_, c_):
            pltpu.make_async_copy(h_hbm.at[srcb[a]], rows[a], gsem[a]).wait()

            @pl.when(k + 2 < KTILE)
            def _():
                pltpu.make_async_copy(src_hbm.at[base + k + 2],
                                      srcb[c_], ssem[c_]).wait()
                pltpu.async_copy(h_hbm.at[srcb[c_]], rows[c_], gsem[c_])

            @pl.when(k + 3 < KTILE)
            def _():
                pltpu.async_copy(src_hbm.at[base + k + 3], srcb[a], ssem[a])

            pltpu.make_async_copy(dst_hbm.at[base + k], dstb[a], dsem[a]).wait()
            pltpu.sync_copy(rows[a], acc_sh.at[dstb[a]], add=True)

            @pl.when(k + 3 < KTILE)
            def _():
                pltpu.async_copy(dst_hbm.at[base + k + 3], dstb[a], dsem[a])

        def body(k, carry):
            m = lax.rem(k, 3)

            @pl.when(m == 0)
            def _():
                step(k, 0, 1, 2)

            @pl.when(m == 1)
            def _():
                step(k, 1, 2, 0)

            @pl.when(m == 2)
            def _():
                step(k, 2, 0, 1)

            return carry

        lax.fori_loop(0, KTILE, body, 0)
        plsc.subcore_barrier()

        @pl.when(s < NS - 1)
        def _():
            pltpu.sync_copy(acc_sh.at[pl.ds(s * OROWS, OROWS)],
                            out_hbm.at[c, pl.ds(s * OROWS, OROWS)])

        @pl.when(s == NS - 1)
        def _():
            pltpu.sync_copy(acc_sh.at[pl.ds((NS - 1) * OROWS, OROWS_LAST)],
                            out_hbm.at[c, pl.ds((NS - 1) * OROWS, OROWS_LAST)])

    return agg_kernel(h, zeros, src, dst)


def _tc_linear(parts, w_t, b, relu):
    blk = 1000
    grid = N // blk

    def body(*refs):
        *a_refs, w_ref, b_ref, o_ref = refs
        h = a_refs[0][...]
        for r in a_refs[1:]:
            h = h + r[...]
        y = jnp.dot(h, w_ref[...], preferred_element_type=jnp.float32) + b_ref[...]
        if relu:
            y = jnp.maximum(y, 0.0)
        o_ref[...] = y

    in_specs = [pl.BlockSpec((blk, D), lambda i: (i, 0)) for _ in parts]
    in_specs += [
        pl.BlockSpec((D, D), lambda i: (0, 0)),
        pl.BlockSpec((1, D), lambda i: (0, 0)),
    ]
    return pl.pallas_call(
        body,
        grid=(grid,),
        in_specs=in_specs,
        out_specs=pl.BlockSpec((blk, D), lambda i: (i, 0)),
        out_shape=jax.ShapeDtypeStruct((N, D), jnp.float32),
    )(*parts, w_t, b)


def kernel(x, edge_index, W_conv, b_conv, W_out, b_out):
    ei = edge_index.astype(jnp.int32)
    pad = E_PAD - E
    src = jnp.concatenate([ei[0], jnp.zeros((pad,), jnp.int32)])
    dst = jnp.concatenate([ei[1], jnp.full((pad,), N, jnp.int32)])
    src = src.reshape(TOT_CHUNKS, CHUNK)
    dst = dst.reshape(TOT_CHUNKS, CHUNK)
    zeros = jnp.zeros((ZROWS, D), jnp.float32)
    wc_t = W_conv.T
    wo_t = W_out.T
    bc = b_conv.reshape(1, D)
    bo = b_out.reshape(1, D)

    h = x
    for _ in range(2):
        parts = _sc_aggregate(h, zeros, src, dst)
        h = _tc_linear((parts[0], parts[1]), wc_t, bc, relu=True)
    return _tc_linear((h,), wo_t, bo, relu=False)


# speedup vs baseline: 2.7417x; 1.5764x over previous
import functools

import jax
import jax.numpy as jnp
from jax import lax
from jax.experimental import pallas as pl
from jax.experimental.pallas import tpu as pltpu
from jax.experimental.pallas import tpu_sc as plsc

N = 10000
E = 320000
D = 128
NC = 2
NS = 16
NW = NC * NS
CHUNK = 128
K0 = 117
K1 = 40
TOT_CHUNKS = NS * (K0 + K1)
E_PAD = TOT_CHUNKS * CHUNK
AGG_ROWS = 10016
ZROWS = 632
ZROWS_LAST = AGG_ROWS - (NS - 1) * ZROWS
OROWS = 632
OROWS_LAST = N - (NS - 1) * OROWS


def _sc_aggregate(h, zeros, src, dst):
    mesh = plsc.VectorSubcoreMesh(core_axis_name="c", subcore_axis_name="s",
                                  num_cores=NC, num_subcores=NS)

    @functools.partial(
        pl.kernel,
        out_type=jax.ShapeDtypeStruct((NC, N, D), jnp.float32),
        mesh=mesh,
        scratch_types=[
            [pltpu.VMEM((CHUNK,), jnp.int32) for _ in range(3)],
            [pltpu.VMEM((CHUNK,), jnp.int32) for _ in range(3)],
            [pltpu.VMEM((CHUNK, D), jnp.float32) for _ in range(3)],
            [pltpu.SemaphoreType.DMA for _ in range(9)],
            pltpu.VMEM_SHARED((AGG_ROWS, D), jnp.float32),
        ],
    )
    def agg_kernel(h_hbm, z_hbm, src_hbm, dst_hbm, out_hbm,
                   srcb, dstb, rows, sems, acc_sh):
        ssem, dsem, gsem = sems[0:3], sems[3:6], sems[6:9]
        c = lax.axis_index("c")
        s = lax.axis_index("s")

        @pl.when(s < NS - 1)
        def _():
            pltpu.sync_copy(z_hbm.at[pl.ds(0, ZROWS)],
                            acc_sh.at[pl.ds(s * ZROWS, ZROWS)])

        @pl.when(s == NS - 1)
        def _():
            pltpu.sync_copy(z_hbm.at[pl.ds(0, ZROWS_LAST)],
                            acc_sh.at[pl.ds((NS - 1) * ZROWS, ZROWS_LAST)])

        plsc.subcore_barrier()

        base = jnp.where(c == 0, s * K0, NS * K0 + s * K1)

        def run_loop(ktile):
            for j in range(3):
                pltpu.async_copy(src_hbm.at[base + j], srcb[j], ssem[j])
                pltpu.async_copy(dst_hbm.at[base + j], dstb[j], dsem[j])
            for j in range(2):
                pltpu.make_async_copy(src_hbm.at[base + j],
                                      srcb[j], ssem[j]).wait()
                pltpu.async_copy(h_hbm.at[srcb[j]], rows[j], gsem[j])

            def step(k, a, b_, c_):
                pltpu.make_async_copy(h_hbm.at[srcb[a]], rows[a], gsem[a]).wait()

                @pl.when(k + 2 < ktile)
                def _():
                    pltpu.make_async_copy(src_hbm.at[base + k + 2],
                                          srcb[c_], ssem[c_]).wait()
                    pltpu.async_copy(h_hbm.at[srcb[c_]], rows[c_], gsem[c_])

                @pl.when(k + 3 < ktile)
                def _():
                    pltpu.async_copy(src_hbm.at[base + k + 3], srcb[a], ssem[a])

                pltpu.make_async_copy(dst_hbm.at[base + k],
                                      dstb[a], dsem[a]).wait()
                pltpu.sync_copy(rows[a], acc_sh.at[dstb[a]], add=True)

                @pl.when(k + 3 < ktile)
                def _():
                    pltpu.async_copy(dst_hbm.at[base + k + 3], dstb[a], dsem[a])

            def body(k, carry):
                m = lax.rem(k, 3)

                @pl.when(m == 0)
                def _():
                    step(k, 0, 1, 2)

                @pl.when(m == 1)
                def _():
                    step(k, 1, 2, 0)

                @pl.when(m == 2)
                def _():
                    step(k, 2, 0, 1)

                return carry

            lax.fori_loop(0, ktile, body, 0)

        @pl.when(c == 0)
        def _():
            run_loop(K0)

        @pl.when(c == 1)
        def _():
            run_loop(K1)

        plsc.subcore_barrier()

        @pl.when(s < NS - 1)
        def _():
            pltpu.sync_copy(acc_sh.at[pl.ds(s * OROWS, OROWS)],
                            out_hbm.at[c, pl.ds(s * OROWS, OROWS)])

        @pl.when(s == NS - 1)
        def _():
            pltpu.sync_copy(acc_sh.at[pl.ds((NS - 1) * OROWS, OROWS_LAST)],
                            out_hbm.at[c, pl.ds((NS - 1) * OROWS, OROWS_LAST)])

    return agg_kernel(h, zeros, src, dst)


def _tc_linear(parts, w_t, b, relu):
    blk = 1000
    grid = N // blk

    def body(*refs):
        *a_refs, w_ref, b_ref, o_ref = refs
        h = a_refs[0][...]
        for r in a_refs[1:]:
            h = h + r[...]
        y = jnp.dot(h, w_ref[...], preferred_element_type=jnp.float32) + b_ref[...]
        if relu:
            y = jnp.maximum(y, 0.0)
        o_ref[...] = y

    in_specs = [pl.BlockSpec((blk, D), lambda i: (i, 0)) for _ in parts]
    in_specs += [
        pl.BlockSpec((D, D), lambda i: (0, 0)),
        pl.BlockSpec((1, D), lambda i: (0, 0)),
    ]
    return pl.pallas_call(
        body,
        grid=(grid,),
        in_specs=in_specs,
        out_specs=pl.BlockSpec((blk, D), lambda i: (i, 0)),
        out_shape=jax.ShapeDtypeStruct((N, D), jnp.float32),
    )(*parts, w_t, b)


def kernel(x, edge_index, W_conv, b_conv, W_out, b_out):
    ei = edge_index.astype(jnp.int32)
    pad = E_PAD - E
    src = jnp.concatenate([ei[0], jnp.zeros((pad,), jnp.int32)])
    dst = jnp.concatenate([ei[1], jnp.full((pad,), N, jnp.int32)])
    src = src.reshape(TOT_CHUNKS, CHUNK)
    dst = dst.reshape(TOT_CHUNKS, CHUNK)
    zeros = jnp.zeros((ZROWS, D), jnp.float32)
    wc_t = W_conv.T
    wo_t = W_out.T
    bc = b_conv.reshape(1, D)
    bo = b_out.reshape(1, D)

    h = x
    for _ in range(2):
        parts = _sc_aggregate(h, zeros, src, dst)
        h = _tc_linear((parts[0], parts[1]), wc_t, bc, relu=True)
    return _tc_linear((h,), wo_t, bo, relu=False)


# speedup vs baseline: 3.2375x; 1.1808x over previous
import functools

import jax
import jax.numpy as jnp
from jax import lax
from jax.experimental import pallas as pl
from jax.experimental.pallas import tpu as pltpu
from jax.experimental.pallas import tpu_sc as plsc

N = 10000
E = 320000
D = 128
NC = 2
NS = 16
NW = NC * NS
CHUNK = 128
TOT_CHUNKS = E // CHUNK
K0 = 117
K1B = (TOT_CHUNKS - NS * K0) // NS
K1N = TOT_CHUNKS - NS * K0 - NS * K1B
K1A = K1B + 1
AGG_ROWS = 10016
ZROWS = 632
ZROWS_LAST = AGG_ROWS - (NS - 1) * ZROWS
OROWS = 632
OROWS_LAST = N - (NS - 1) * OROWS


def _sc_aggregate(h, zeros, src, dst):
    mesh = plsc.VectorSubcoreMesh(core_axis_name="c", subcore_axis_name="s",
                                  num_cores=NC, num_subcores=NS)

    @functools.partial(
        pl.kernel,
        out_type=jax.ShapeDtypeStruct((NC, N, D), jnp.float32),
        mesh=mesh,
        scratch_types=[
            [pltpu.VMEM((CHUNK,), jnp.int32) for _ in range(3)],
            [pltpu.VMEM((CHUNK,), jnp.int32) for _ in range(3)],
            [pltpu.VMEM((CHUNK, D), jnp.float32) for _ in range(3)],
            [pltpu.SemaphoreType.DMA for _ in range(9)],
            pltpu.VMEM_SHARED((AGG_ROWS, D), jnp.float32),
        ],
    )
    def agg_kernel(h_hbm, z_hbm, src_hbm, dst_hbm, out_hbm,
                   srcb, dstb, rows, sems, acc_sh):
        ssem, dsem, gsem = sems[0:3], sems[3:6], sems[6:9]
        c = lax.axis_index("c")
        s = lax.axis_index("s")

        @pl.when(s < NS - 1)
        def _():
            pltpu.sync_copy(z_hbm.at[pl.ds(0, ZROWS)],
                            acc_sh.at[pl.ds(s * ZROWS, ZROWS)])

        @pl.when(s == NS - 1)
        def _():
            pltpu.sync_copy(z_hbm.at[pl.ds(0, ZROWS_LAST)],
                            acc_sh.at[pl.ds((NS - 1) * ZROWS, ZROWS_LAST)])

        plsc.subcore_barrier()

        base = jnp.where(c == 0, s * K0,
                         NS * K0 + s * K1B + jnp.minimum(s, K1N))

        def run_loop(ktile):
            for j in range(3):
                pltpu.async_copy(src_hbm.at[base + j], srcb[j], ssem[j])
                pltpu.async_copy(dst_hbm.at[base + j], dstb[j], dsem[j])
            for j in range(2):
                pltpu.make_async_copy(src_hbm.at[base + j],
                                      srcb[j], ssem[j]).wait()
                pltpu.async_copy(h_hbm.at[srcb[j]], rows[j], gsem[j])

            def step(k, a, b_, c_):
                pltpu.make_async_copy(h_hbm.at[srcb[a]], rows[a], gsem[a]).wait()

                @pl.when(k + 2 < ktile)
                def _():
                    pltpu.make_async_copy(src_hbm.at[base + k + 2],
                                          srcb[c_], ssem[c_]).wait()
                    pltpu.async_copy(h_hbm.at[srcb[c_]], rows[c_], gsem[c_])

                @pl.when(k + 3 < ktile)
                def _():
                    pltpu.async_copy(src_hbm.at[base + k + 3], srcb[a], ssem[a])

                pltpu.make_async_copy(dst_hbm.at[base + k],
                                      dstb[a], dsem[a]).wait()
                pltpu.sync_copy(rows[a], acc_sh.at[dstb[a]], add=True)

                @pl.when(k + 3 < ktile)
                def _():
                    pltpu.async_copy(dst_hbm.at[base + k + 3], dstb[a], dsem[a])

            def body(k, carry):
                m = lax.rem(k, 3)

                @pl.when(m == 0)
                def _():
                    step(k, 0, 1, 2)

                @pl.when(m == 1)
                def _():
                    step(k, 1, 2, 0)

                @pl.when(m == 2)
                def _():
                    step(k, 2, 0, 1)

                return carry

            lax.fori_loop(0, ktile, body, 0)

        @pl.when(c == 0)
        def _():
            run_loop(K0)

        @pl.when(jnp.logical_and(c == 1, s < K1N))
        def _():
            run_loop(K1A)

        @pl.when(jnp.logical_and(c == 1, s >= K1N))
        def _():
            run_loop(K1B)

        plsc.subcore_barrier()

        @pl.when(s < NS - 1)
        def _():
            pltpu.sync_copy(acc_sh.at[pl.ds(s * OROWS, OROWS)],
                            out_hbm.at[c, pl.ds(s * OROWS, OROWS)])

        @pl.when(s == NS - 1)
        def _():
            pltpu.sync_copy(acc_sh.at[pl.ds((NS - 1) * OROWS, OROWS_LAST)],
                            out_hbm.at[c, pl.ds((NS - 1) * OROWS, OROWS_LAST)])

    return agg_kernel(h, zeros, src, dst)


_BLK = 1000


def _part_specs():
    return [pl.BlockSpec((1, _BLK, D), lambda i: (0, i, 0)),
            pl.BlockSpec((1, _BLK, D), lambda i: (1, i, 0))]


def _w_spec():
    return pl.BlockSpec((D, D), lambda i: (0, 0))


def _b_spec():
    return pl.BlockSpec((1, D), lambda i: (0, 0))


def _tc_layer(parts, w_t, b):
    def body(a0_ref, a1_ref, w_ref, b_ref, o_ref):
        h = a0_ref[0] + a1_ref[0]
        y = jnp.dot(h, w_ref[...], preferred_element_type=jnp.float32) + b_ref[...]
        o_ref[...] = jnp.maximum(y, 0.0)

    return pl.pallas_call(
        body,
        grid=(N // _BLK,),
        in_specs=_part_specs() + [_w_spec(), _b_spec()],
        out_specs=pl.BlockSpec((_BLK, D), lambda i: (i, 0)),
        out_shape=jax.ShapeDtypeStruct((N, D), jnp.float32),
    )(parts, parts, w_t, b)


def _tc_final(parts, wc_t, bc, wo_t, bo):
    def body(a0_ref, a1_ref, wc_ref, bc_ref, wo_ref, bo_ref, o_ref):
        h = a0_ref[0] + a1_ref[0]
        t = jnp.dot(h, wc_ref[...], preferred_element_type=jnp.float32)
        t = jnp.maximum(t + bc_ref[...], 0.0)
        y = jnp.dot(t, wo_ref[...], preferred_element_type=jnp.float32)
        o_ref[...] = y + bo_ref[...]

    return pl.pallas_call(
        body,
        grid=(N // _BLK,),
        in_specs=_part_specs() + [_w_spec(), _b_spec(), _w_spec(), _b_spec()],
        out_specs=pl.BlockSpec((_BLK, D), lambda i: (i, 0)),
        out_shape=jax.ShapeDtypeStruct((N, D), jnp.float32),
    )(parts, parts, wc_t, bc, wo_t, bo)


def kernel(x, edge_index, W_conv, b_conv, W_out, b_out):
    ei = edge_index.astype(jnp.int32)
    src = ei[0].reshape(TOT_CHUNKS, CHUNK)
    dst = ei[1].reshape(TOT_CHUNKS, CHUNK)
    zeros = jnp.zeros((ZROWS, D), jnp.float32)
    wc_t = W_conv.T
    wo_t = W_out.T
    bc = b_conv.reshape(1, D)
    bo = b_out.reshape(1, D)

    parts1 = _sc_aggregate(x, zeros, src, dst)
    h1 = _tc_layer(parts1, wc_t, bc)
    parts2 = _sc_aggregate(h1, zeros, src, dst)
    return _tc_final(parts2, wc_t, bc, wo_t, bo)


# speedup vs baseline: 3.2712x; 1.0104x over previous
import functools

import jax
import jax.numpy as jnp
from jax import lax
from jax.experimental import pallas as pl
from jax.experimental.pallas import tpu as pltpu
from jax.experimental.pallas import tpu_sc as plsc

N = 10000
E = 320000
D = 128
NC = 2
NS = 16
NW = NC * NS
CHUNK = 128
TOT_CHUNKS = E // CHUNK
K0 = 117
K1B = (TOT_CHUNKS - NS * K0) // NS
K1N = TOT_CHUNKS - NS * K0 - NS * K1B
K1A = K1B + 1
AGG_ROWS = 10016
ZROWS = 632
ZROWS_LAST = AGG_ROWS - (NS - 1) * ZROWS
OROWS = 632
OROWS_LAST = N - (NS - 1) * OROWS


def _sc_aggregate(h, src, dst):
    mesh = plsc.VectorSubcoreMesh(core_axis_name="c", subcore_axis_name="s",
                                  num_cores=NC, num_subcores=NS)

    @functools.partial(
        pl.kernel,
        out_type=jax.ShapeDtypeStruct((NC, N, D), jnp.float32),
        mesh=mesh,
        scratch_types=[
            [pltpu.VMEM((CHUNK,), jnp.int32) for _ in range(3)],
            [pltpu.VMEM((CHUNK,), jnp.int32) for _ in range(3)],
            [pltpu.VMEM((CHUNK, D), jnp.float32) for _ in range(3)],
            [pltpu.SemaphoreType.DMA for _ in range(9)],
            pltpu.VMEM_SHARED((AGG_ROWS, D), jnp.float32),
        ],
    )
    def agg_kernel(h_hbm, src_hbm, dst_hbm, out_hbm,
                   srcb, dstb, rows, sems, acc_sh):
        ssem, dsem, gsem = sems[0:3], sems[3:6], sems[6:9]
        c = lax.axis_index("c")
        s = lax.axis_index("s")

        zv = jnp.zeros((16,), jnp.float32)

        def zbody(i, carry):
            rows[0][i // 8, pl.ds(lax.rem(i, 8) * 16, 16)] = zv
            return carry

        lax.fori_loop(0, CHUNK * 8, zbody, 0)
        for j in range(ZROWS // CHUNK):
            pltpu.sync_copy(rows[0],
                            acc_sh.at[pl.ds(s * ZROWS + j * CHUNK, CHUNK)])
        ztail = ZROWS - (ZROWS // CHUNK) * CHUNK
        ztail_last = ZROWS_LAST - (ZROWS // CHUNK) * CHUNK

        @pl.when(s < NS - 1)
        def _():
            pltpu.sync_copy(rows[0].at[pl.ds(0, ztail)],
                            acc_sh.at[pl.ds(s * ZROWS + ZROWS - ztail, ztail)])

        @pl.when(s == NS - 1)
        def _():
            pltpu.sync_copy(
                rows[0].at[pl.ds(0, ztail_last)],
                acc_sh.at[pl.ds(s * ZROWS + ZROWS_LAST - ztail_last,
                                ztail_last)])

        plsc.subcore_barrier()

        base = jnp.where(c == 0, s * K0,
                         NS * K0 + s * K1B + jnp.minimum(s, K1N))

        def run_loop(ktile):
            for j in range(3):
                pltpu.async_copy(src_hbm.at[base + j], srcb[j], ssem[j])
                pltpu.async_copy(dst_hbm.at[base + j], dstb[j], dsem[j])
            for j in range(2):
                pltpu.make_async_copy(src_hbm.at[base + j],
                                      srcb[j], ssem[j]).wait()
                pltpu.async_copy(h_hbm.at[srcb[j]], rows[j], gsem[j])

            def step(k, a, b_, c_):
                pltpu.make_async_copy(h_hbm.at[srcb[a]], rows[a], gsem[a]).wait()

                @pl.when(k + 2 < ktile)
                def _():
                    pltpu.make_async_copy(src_hbm.at[base + k + 2],
                                          srcb[c_], ssem[c_]).wait()
                    pltpu.async_copy(h_hbm.at[srcb[c_]], rows[c_], gsem[c_])

                @pl.when(k + 3 < ktile)
                def _():
                    pltpu.async_copy(src_hbm.at[base + k + 3], srcb[a], ssem[a])

                pltpu.make_async_copy(dst_hbm.at[base + k],
                                      dstb[a], dsem[a]).wait()
                pltpu.sync_copy(rows[a], acc_sh.at[dstb[a]], add=True)

                @pl.when(k + 3 < ktile)
                def _():
                    pltpu.async_copy(dst_hbm.at[base + k + 3], dstb[a], dsem[a])

            def body(k, carry):
                m = lax.rem(k, 3)

                @pl.when(m == 0)
                def _():
                    step(k, 0, 1, 2)

                @pl.when(m == 1)
                def _():
                    step(k, 1, 2, 0)

                @pl.when(m == 2)
                def _():
                    step(k, 2, 0, 1)

                return carry

            lax.fori_loop(0, ktile, body, 0)

        @pl.when(c == 0)
        def _():
            run_loop(K0)

        @pl.when(jnp.logical_and(c == 1, s < K1N))
        def _():
            run_loop(K1A)

        @pl.when(jnp.logical_and(c == 1, s >= K1N))
        def _():
            run_loop(K1B)

        plsc.subcore_barrier()

        @pl.when(s < NS - 1)
        def _():
            pltpu.sync_copy(acc_sh.at[pl.ds(s * OROWS, OROWS)],
                            out_hbm.at[c, pl.ds(s * OROWS, OROWS)])

        @pl.when(s == NS - 1)
        def _():
            pltpu.sync_copy(acc_sh.at[pl.ds((NS - 1) * OROWS, OROWS_LAST)],
                            out_hbm.at[c, pl.ds((NS - 1) * OROWS, OROWS_LAST)])

    return agg_kernel(h, src, dst)


_BLK = 1000


def _part_specs():
    return [pl.BlockSpec((1, _BLK, D), lambda i: (0, i, 0)),
            pl.BlockSpec((1, _BLK, D), lambda i: (1, i, 0))]


def _w_spec():
    return pl.BlockSpec((D, D), lambda i: (0, 0))


def _b_spec():
    return pl.BlockSpec((1, D), lambda i: (0, 0))


def _tc_layer(parts, w_t, b):
    def body(a0_ref, a1_ref, w_ref, b_ref, o_ref):
        h = a0_ref[0] + a1_ref[0]
        y = jnp.dot(h, w_ref[...], preferred_element_type=jnp.float32) + b_ref[...]
        o_ref[...] = jnp.maximum(y, 0.0)

    return pl.pallas_call(
        body,
        grid=(N // _BLK,),
        in_specs=_part_specs() + [_w_spec(), _b_spec()],
        out_specs=pl.BlockSpec((_BLK, D), lambda i: (i, 0)),
        out_shape=jax.ShapeDtypeStruct((N, D), jnp.float32),
    )(parts, parts, w_t, b)


def _tc_final(parts, wc_t, bc, wo_t, bo):
    def body(a0_ref, a1_ref, wc_ref, bc_ref, wo_ref, bo_ref, o_ref):
        h = a0_ref[0] + a1_ref[0]
        t = jnp.dot(h, wc_ref[...], preferred_element_type=jnp.float32)
        t = jnp.maximum(t + bc_ref[...], 0.0)
        y = jnp.dot(t, wo_ref[...], preferred_element_type=jnp.float32)
        o_ref[...] = y + bo_ref[...]

    return pl.pallas_call(
        body,
        grid=(N // _BLK,),
        in_specs=_part_specs() + [_w_spec(), _b_spec(), _w_spec(), _b_spec()],
        out_specs=pl.BlockSpec((_BLK, D), lambda i: (i, 0)),
        out_shape=jax.ShapeDtypeStruct((N, D), jnp.float32),
    )(parts, parts, wc_t, bc, wo_t, bo)


def kernel(x, edge_index, W_conv, b_conv, W_out, b_out):
    ei = edge_index.astype(jnp.int32)
    src = ei[0].reshape(TOT_CHUNKS, CHUNK)
    dst = ei[1].reshape(TOT_CHUNKS, CHUNK)
    wc_t = W_conv.T
    wo_t = W_out.T
    bc = b_conv.reshape(1, D)
    bo = b_out.reshape(1, D)

    parts1 = _sc_aggregate(x, src, dst)
    h1 = _tc_layer(parts1, wc_t, bc)
    parts2 = _sc_aggregate(h1, src, dst)
    return _tc_final(parts2, wc_t, bc, wo_t, bo)


# speedup vs baseline: 3.9767x; 1.2157x over previous
import functools

import jax
import jax.numpy as jnp
from jax import lax
from jax.experimental import pallas as pl
from jax.experimental.pallas import tpu as pltpu
from jax.experimental.pallas import tpu_sc as plsc

N = 10000
E = 320000
D = 128
NC = 2
NS = 16
NW = NC * NS
CHUNK = 128
TOT_CHUNKS = E // CHUNK
K0 = 88
K1B = (TOT_CHUNKS - NS * K0) // NS
K1N = TOT_CHUNKS - NS * K0 - NS * K1B
K1A = K1B + 1
AGG_ROWS = 10016
ZROWS = 632
ZROWS_LAST = AGG_ROWS - (NS - 1) * ZROWS
OROWS = 632
OROWS_LAST = N - (NS - 1) * OROWS


def _sc_aggregate(h, src, dst):
    mesh = plsc.VectorSubcoreMesh(core_axis_name="c", subcore_axis_name="s",
                                  num_cores=NC, num_subcores=NS)

    @functools.partial(
        pl.kernel,
        out_type=jax.ShapeDtypeStruct((NC, N, D), jnp.float32),
        mesh=mesh,
        scratch_types=[
            [pltpu.VMEM((CHUNK,), jnp.int32) for _ in range(3)],
            [pltpu.VMEM((CHUNK,), jnp.int32) for _ in range(3)],
            [pltpu.VMEM((CHUNK, D), jnp.float32) for _ in range(3)],
            [pltpu.SemaphoreType.DMA for _ in range(9)],
            pltpu.VMEM_SHARED((AGG_ROWS, D), jnp.float32),
        ],
    )
    def agg_kernel(h_hbm, src_hbm, dst_hbm, out_hbm,
                   srcb, dstb, rows, sems, acc_sh):
        ssem, dsem, gsem = sems[0:3], sems[3:6], sems[6:9]
        c = lax.axis_index("c")
        s = lax.axis_index("s")

        zv = jnp.zeros((16,), jnp.float32)

        def zbody(i, carry):
            rows[0][i // 8, pl.ds(lax.rem(i, 8) * 16, 16)] = zv
            return carry

        lax.fori_loop(0, CHUNK * 8, zbody, 0)
        for j in range(ZROWS // CHUNK):
            pltpu.sync_copy(rows[0],
                            acc_sh.at[pl.ds(s * ZROWS + j * CHUNK, CHUNK)])
        ztail = ZROWS - (ZROWS // CHUNK) * CHUNK
        ztail_last = ZROWS_LAST - (ZROWS // CHUNK) * CHUNK

        @pl.when(s < NS - 1)
        def _():
            pltpu.sync_copy(rows[0].at[pl.ds(0, ztail)],
                            acc_sh.at[pl.ds(s * ZROWS + ZROWS - ztail, ztail)])

        @pl.when(s == NS - 1)
        def _():
            pltpu.sync_copy(
                rows[0].at[pl.ds(0, ztail_last)],
                acc_sh.at[pl.ds(s * ZROWS + ZROWS_LAST - ztail_last,
                                ztail_last)])

        plsc.subcore_barrier()

        base = jnp.where(c == 0, s * K0,
                         NS * K0 + s * K1B + jnp.minimum(s, K1N))

        def run_loop(ktile):
            for j in range(3):
                pltpu.async_copy(src_hbm.at[base + j], srcb[j], ssem[j])
                pltpu.async_copy(dst_hbm.at[base + j], dstb[j], dsem[j])
            for j in range(2):
                pltpu.make_async_copy(src_hbm.at[base + j],
                                      srcb[j], ssem[j]).wait()
                pltpu.async_copy(h_hbm.at[srcb[j]], rows[j], gsem[j])

            def step(k, a, b_, c_):
                pltpu.make_async_copy(h_hbm.at[srcb[a]], rows[a], gsem[a]).wait()

                @pl.when(k + 2 < ktile)
                def _():
                    pltpu.make_async_copy(src_hbm.at[base + k + 2],
                                          srcb[c_], ssem[c_]).wait()
                    pltpu.async_copy(h_hbm.at[srcb[c_]], rows[c_], gsem[c_])

                @pl.when(k + 3 < ktile)
                def _():
                    pltpu.async_copy(src_hbm.at[base + k + 3], srcb[a], ssem[a])

                pltpu.make_async_copy(dst_hbm.at[base + k],
                                      dstb[a], dsem[a]).wait()
                pltpu.sync_copy(rows[a], acc_sh.at[dstb[a]], add=True)

                @pl.when(k + 3 < ktile)
                def _():
                    pltpu.async_copy(dst_hbm.at[base + k + 3], dstb[a], dsem[a])

            def body(k, carry):
                m = lax.rem(k, 3)

                @pl.when(m == 0)
                def _():
                    step(k, 0, 1, 2)

                @pl.when(m == 1)
                def _():
                    step(k, 1, 2, 0)

                @pl.when(m == 2)
                def _():
                    step(k, 2, 0, 1)

                return carry

            lax.fori_loop(0, ktile, body, 0)

        @pl.when(c == 0)
        def _():
            run_loop(K0)

        @pl.when(jnp.logical_and(c == 1, s < K1N))
        def _():
            run_loop(K1A)

        @pl.when(jnp.logical_and(c == 1, s >= K1N))
        def _():
            run_loop(K1B)

        plsc.subcore_barrier()

        @pl.when(s < NS - 1)
        def _():
            pltpu.sync_copy(acc_sh.at[pl.ds(s * OROWS, OROWS)],
                            out_hbm.at[c, pl.ds(s * OROWS, OROWS)])

        @pl.when(s == NS - 1)
        def _():
            pltpu.sync_copy(acc_sh.at[pl.ds((NS - 1) * OROWS, OROWS_LAST)],
                            out_hbm.at[c, pl.ds((NS - 1) * OROWS, OROWS_LAST)])

    return agg_kernel(h, src, dst)


_BLK = 1000


def _part_specs():
    return [pl.BlockSpec((1, _BLK, D), lambda i: (0, i, 0)),
            pl.BlockSpec((1, _BLK, D), lambda i: (1, i, 0))]


def _w_spec():
    return pl.BlockSpec((D, D), lambda i: (0, 0))


def _b_spec():
    return pl.BlockSpec((1, D), lambda i: (0, 0))


def _tc_layer(parts, w_t, b):
    def body(a0_ref, a1_ref, w_ref, b_ref, o_ref):
        h = a0_ref[0] + a1_ref[0]
        y = jnp.dot(h, w_ref[...], preferred_element_type=jnp.float32) + b_ref[...]
        o_ref[...] = jnp.maximum(y, 0.0)

    return pl.pallas_call(
        body,
        grid=(N // _BLK,),
        in_specs=_part_specs() + [_w_spec(), _b_spec()],
        out_specs=pl.BlockSpec((_BLK, D), lambda i: (i, 0)),
        out_shape=jax.ShapeDtypeStruct((N, D), jnp.float32),
    )(parts, parts, w_t, b)


def _tc_final(parts, wc_t, bc, wo_t, bo):
    def body(a0_ref, a1_ref, wc_ref, bc_ref, wo_ref, bo_ref, o_ref):
        h = a0_ref[0] + a1_ref[0]
        t = jnp.dot(h, wc_ref[...], preferred_element_type=jnp.float32)
        t = jnp.maximum(t + bc_ref[...], 0.0)
        y = jnp.dot(t, wo_ref[...], preferred_element_type=jnp.float32)
        o_ref[...] = y + bo_ref[...]

    return pl.pallas_call(
        body,
        grid=(N // _BLK,),
        in_specs=_part_specs() + [_w_spec(), _b_spec(), _w_spec(), _b_spec()],
        out_specs=pl.BlockSpec((_BLK, D), lambda i: (i, 0)),
        out_shape=jax.ShapeDtypeStruct((N, D), jnp.float32),
    )(parts, parts, wc_t, bc, wo_t, bo)


def kernel(x, edge_index, W_conv, b_conv, W_out, b_out):
    ei = edge_index.astype(jnp.int32)
    src = ei[0].reshape(TOT_CHUNKS, CHUNK)
    dst = ei[1].reshape(TOT_CHUNKS, CHUNK)
    wc_t = W_conv.T
    wo_t = W_out.T
    bc = b_conv.reshape(1, D)
    bo = b_out.reshape(1, D)

    parts1 = _sc_aggregate(x, src, dst)
    h1 = _tc_layer(parts1, wc_t, bc)
    parts2 = _sc_aggregate(h1, src, dst)
    return _tc_final(parts2, wc_t, bc, wo_t, bo)


# speedup vs baseline: 4.5200x; 1.1366x over previous
import functools

import jax
import jax.numpy as jnp
from jax import lax
from jax.experimental import pallas as pl
from jax.experimental.pallas import tpu as pltpu
from jax.experimental.pallas import tpu_sc as plsc

N = 10000
E = 320000
D = 128
NC = 2
NS = 16
NW = NC * NS
CHUNK = 128
TOT_CHUNKS = E // CHUNK
K0 = 80
K1B = (TOT_CHUNKS - NS * K0) // NS
K1N = TOT_CHUNKS - NS * K0 - NS * K1B
K1A = K1B + 1
AGG_ROWS = 10016
ZROWS = 632
ZROWS_LAST = AGG_ROWS - (NS - 1) * ZROWS
OROWS = 632
OROWS_LAST = N - (NS - 1) * OROWS


def _sc_aggregate(h, ei):
    mesh = plsc.VectorSubcoreMesh(core_axis_name="c", subcore_axis_name="s",
                                  num_cores=NC, num_subcores=NS)

    @functools.partial(
        pl.kernel,
        out_type=jax.ShapeDtypeStruct((NC, N, D), jnp.float32),
        mesh=mesh,
        scratch_types=[
            [pltpu.VMEM((CHUNK,), jnp.int32) for _ in range(3)],
            [pltpu.VMEM((CHUNK,), jnp.int32) for _ in range(3)],
            [pltpu.VMEM((CHUNK, D), jnp.float32) for _ in range(3)],
            [pltpu.SemaphoreType.DMA for _ in range(9)],
            pltpu.VMEM_SHARED((AGG_ROWS, D), jnp.float32),
        ],
    )
    def agg_kernel(h_hbm, ei_hbm, out_hbm,
                   srcb, dstb, rows, sems, acc_sh):
        ssem, dsem, gsem = sems[0:3], sems[3:6], sems[6:9]
        c = lax.axis_index("c")
        s = lax.axis_index("s")

        zv = jnp.zeros((16,), jnp.float32)

        def zbody(i, carry):
            for j in range(8):
                rows[0][i, pl.ds(j * 16, 16)] = zv
            return carry

        lax.fori_loop(0, CHUNK, zbody, 0)
        for j in range(ZROWS // CHUNK):
            pltpu.sync_copy(rows[0],
                            acc_sh.at[pl.ds(s * ZROWS + j * CHUNK, CHUNK)])
        ztail = ZROWS - (ZROWS // CHUNK) * CHUNK
        ztail_last = ZROWS_LAST - (ZROWS // CHUNK) * CHUNK

        @pl.when(s < NS - 1)
        def _():
            pltpu.sync_copy(rows[0].at[pl.ds(0, ztail)],
                            acc_sh.at[pl.ds(s * ZROWS + ZROWS - ztail, ztail)])

        @pl.when(s == NS - 1)
        def _():
            pltpu.sync_copy(
                rows[0].at[pl.ds(0, ztail_last)],
                acc_sh.at[pl.ds(s * ZROWS + ZROWS_LAST - ztail_last,
                                ztail_last)])

        plsc.subcore_barrier()

        base = jnp.where(c == 0, s * K0,
                         NS * K0 + s * K1B + jnp.minimum(s, K1N))

        def run_loop(ktile):
            for j in range(3):
                pltpu.async_copy(ei_hbm.at[0, base + j], srcb[j], ssem[j])
                pltpu.async_copy(ei_hbm.at[1, base + j], dstb[j], dsem[j])
            for j in range(2):
                pltpu.make_async_copy(ei_hbm.at[0, base + j],
                                      srcb[j], ssem[j]).wait()
                pltpu.async_copy(h_hbm.at[srcb[j]], rows[j], gsem[j])

            def step(k, a, b_, c_):
                pltpu.make_async_copy(h_hbm.at[srcb[a]], rows[a], gsem[a]).wait()

                @pl.when(k + 2 < ktile)
                def _():
                    pltpu.make_async_copy(ei_hbm.at[0, base + k + 2],
                                          srcb[c_], ssem[c_]).wait()
                    pltpu.async_copy(h_hbm.at[srcb[c_]], rows[c_], gsem[c_])

                @pl.when(k + 3 < ktile)
                def _():
                    pltpu.async_copy(ei_hbm.at[0, base + k + 3], srcb[a], ssem[a])

                pltpu.make_async_copy(ei_hbm.at[1, base + k],
                                      dstb[a], dsem[a]).wait()
                pltpu.sync_copy(rows[a], acc_sh.at[dstb[a]], add=True)

                @pl.when(k + 3 < ktile)
                def _():
                    pltpu.async_copy(ei_hbm.at[1, base + k + 3], dstb[a], dsem[a])

            def body(k, carry):
                m = lax.rem(k, 3)

                @pl.when(m == 0)
                def _():
                    step(k, 0, 1, 2)

                @pl.when(m == 1)
                def _():
                    step(k, 1, 2, 0)

                @pl.when(m == 2)
                def _():
                    step(k, 2, 0, 1)

                return carry

            lax.fori_loop(0, ktile, body, 0)

        @pl.when(c == 0)
        def _():
            run_loop(K0)

        @pl.when(jnp.logical_and(c == 1, s < K1N))
        def _():
            run_loop(K1A)

        @pl.when(jnp.logical_and(c == 1, s >= K1N))
        def _():
            run_loop(K1B)

        plsc.subcore_barrier()

        @pl.when(s < NS - 1)
        def _():
            pltpu.sync_copy(acc_sh.at[pl.ds(s * OROWS, OROWS)],
                            out_hbm.at[c, pl.ds(s * OROWS, OROWS)])

        @pl.when(s == NS - 1)
        def _():
            pltpu.sync_copy(acc_sh.at[pl.ds((NS - 1) * OROWS, OROWS_LAST)],
                            out_hbm.at[c, pl.ds((NS - 1) * OROWS, OROWS_LAST)])

    return agg_kernel(h, ei)


_BLK = 1000


def _part_specs():
    return [pl.BlockSpec((1, _BLK, D), lambda i: (0, i, 0)),
            pl.BlockSpec((1, _BLK, D), lambda i: (1, i, 0))]


def _w_spec():
    return pl.BlockSpec((D, D), lambda i: (0, 0))


def _b_spec():
    return pl.BlockSpec((1, D), lambda i: (0, 0))


def _tc_layer(parts, w_t, b):
    def body(a0_ref, a1_ref, w_ref, b_ref, o_ref):
        h = a0_ref[0] + a1_ref[0]
        y = jnp.dot(h, w_ref[...], preferred_element_type=jnp.float32) + b_ref[...]
        o_ref[...] = jnp.maximum(y, 0.0)

    return pl.pallas_call(
        body,
        grid=(N // _BLK,),
        in_specs=_part_specs() + [_w_spec(), _b_spec()],
        out_specs=pl.BlockSpec((_BLK, D), lambda i: (i, 0)),
        out_shape=jax.ShapeDtypeStruct((N, D), jnp.float32),
    )(parts, parts, w_t, b)


def _tc_final(parts, wc_t, bc, wo_t, bo):
    def body(a0_ref, a1_ref, wc_ref, bc_ref, wo_ref, bo_ref, o_ref):
        h = a0_ref[0] + a1_ref[0]
        t = jnp.dot(h, wc_ref[...], preferred_element_type=jnp.float32)
        t = jnp.maximum(t + bc_ref[...], 0.0)
        y = jnp.dot(t, wo_ref[...], preferred_element_type=jnp.float32)
        o_ref[...] = y + bo_ref[...]

    return pl.pallas_call(
        body,
        grid=(N // _BLK,),
        in_specs=_part_specs() + [_w_spec(), _b_spec(), _w_spec(), _b_spec()],
        out_specs=pl.BlockSpec((_BLK, D), lambda i: (i, 0)),
        out_shape=jax.ShapeDtypeStruct((N, D), jnp.float32),
    )(parts, parts, wc_t, bc, wo_t, bo)


def kernel(x, edge_index, W_conv, b_conv, W_out, b_out):
    ei = edge_index.astype(jnp.int32).reshape(2, TOT_CHUNKS, CHUNK)
    wc_t = W_conv.T
    wo_t = W_out.T
    bc = b_conv.reshape(1, D)
    bo = b_out.reshape(1, D)

    parts1 = _sc_aggregate(x, ei)
    h1 = _tc_layer(parts1, wc_t, bc)
    parts2 = _sc_aggregate(h1, ei)
    return _tc_final(parts2, wc_t, bc, wo_t, bo)
